# Initial kernel scaffold; baseline (speedup 1.0000x reference)
import jax
import jax.numpy as jnp
from jax.experimental import pallas as pl


def kernel(pos_1, pos_2, batch, W1a, b1a, gamma1, beta1, W1b, b1b, W2, b2, Wlin, blin, Wm1, bm1, Wm2, bm2):
    raise NotImplementedError("write your pallas kernel here")


# speedup vs baseline: 6.3396x; 6.3396x over previous
import functools

import jax
import jax.numpy as jnp
from jax import lax

_PH = lax.Precision.HIGHEST
from jax.experimental import pallas as pl
from jax.experimental.pallas import tpu as pltpu
from jax.experimental.pallas import tpu_sc as plsc

N = 8192
K = 20
G = 8
R = 128
NB = N // R

_f32 = jnp.float32
_i32 = jnp.int32


def _knn_body(xb_ref, xf_ref, xt_ref, bc_ref, br_ref, wq_ref, bq_ref, wd_ref,
              idx_ref, a_ref, b_ref):
    D = xf_ref.shape[1]
    xb = xb_ref[...]
    xf = xf_ref[...]
    a_ref[...] = jnp.dot(xb, wq_ref[...], preferred_element_type=_f32, precision=_PH) + bq_ref[...]
    b_ref[...] = jnp.dot(xb, wd_ref[...], preferred_element_type=_f32, precision=_PH)

    sqr = xt_ref[0:1, :] * xt_ref[0:1, :]
    for c in range(1, D):
        t = xt_ref[c:c + 1, :]
        sqr = sqr + t * t
    sqb = jnp.sum(xb * xb, axis=1, keepdims=True)
    xx = lax.dot_general(xb, xf, (((1,), (1,)), ((), ())),
                         preferred_element_type=_f32)
    d = sqb + sqr - 2.0 * xx
    d = jnp.where(bc_ref[...] != br_ref[...], jnp.inf, d)
    colid = lax.broadcasted_iota(_i32, (R, N), 1)
    cols = []
    for _ in range(K):
        m = jnp.min(d, axis=1, keepdims=True)
        cand = jnp.where(d == m, colid, N)
        a = jnp.min(cand, axis=1, keepdims=True)
        cols.append(a)
        d = jnp.where(colid == a, jnp.inf, d)
    idx_ref[...] = jnp.concatenate(cols, axis=1)


def _knn_feats(x, bcol, brow, Wq, bq, Wd):
    D = x.shape[1]
    C = Wd.shape[1]
    return pl.pallas_call(
        _knn_body,
        grid=(NB,),
        in_specs=[
            pl.BlockSpec((R, D), lambda i: (i, 0)),
            pl.BlockSpec((N, D), lambda i: (0, 0)),
            pl.BlockSpec((D, N), lambda i: (0, 0)),
            pl.BlockSpec((R, 1), lambda i: (i, 0)),
            pl.BlockSpec((1, N), lambda i: (0, 0)),
            pl.BlockSpec((D, C), lambda i: (0, 0)),
            pl.BlockSpec((1, C), lambda i: (0, 0)),
            pl.BlockSpec((D, C), lambda i: (0, 0)),
        ],
        out_specs=[
            pl.BlockSpec((R, K), lambda i: (i, 0)),
            pl.BlockSpec((R, C), lambda i: (i, 0)),
            pl.BlockSpec((R, C), lambda i: (i, 0)),
        ],
        out_shape=[
            jax.ShapeDtypeStruct((N, K), _i32),
            jax.ShapeDtypeStruct((N, C), _f32),
            jax.ShapeDtypeStruct((N, C), _f32),
        ],
    )(x, x, x.T, bcol, brow, Wq, bq, Wd)


def _sc_gather(table, idx_flat):
    D = table.shape[1]
    B = idx_flat.shape[0]
    NC, NS = 2, 16
    NW = NC * NS
    CH = 128
    b_per_w = B // NW
    n_ch = b_per_w // CH
    mesh = plsc.VectorSubcoreMesh(core_axis_name="c", subcore_axis_name="s")

    @functools.partial(
        pl.kernel, mesh=mesh,
        out_type=jax.ShapeDtypeStruct((B, D), _f32),
        scratch_types=[
            pltpu.VMEM((CH,), _i32),
            pltpu.VMEM((CH, D), _f32),
            pltpu.SemaphoreType.DMA,
        ],
    )
    def gk(table_hbm, idx_hbm, out_hbm, idx_v, rows_v, sem):
        wid = lax.axis_index("s") * NC + lax.axis_index("c")
        base = wid * b_per_w

        def body(i, carry):
            off = base + i * CH
            pltpu.sync_copy(idx_hbm.at[pl.ds(off, CH)], idx_v)
            pltpu.async_copy(table_hbm.at[idx_v], rows_v, sem).wait()
            pltpu.sync_copy(rows_v, out_hbm.at[pl.ds(off, CH)])
            return carry

        lax.fori_loop(0, n_ch, body, 0)

    return gk(table, idx_flat)


def _stats_body(a_ref, bg_ref, s_ref):
    A = a_ref[...]
    s = jnp.zeros((1, 128), _f32)
    ss = jnp.zeros((1, 128), _f32)
    for k in range(K):
        h = A + bg_ref[k]
        s = s + jnp.sum(h, axis=0, keepdims=True)
        ss = ss + jnp.sum(h * h, axis=0, keepdims=True)
    contrib = jnp.concatenate([s, ss, jnp.zeros((6, 128), _f32)], axis=0)

    @pl.when(pl.program_id(0) == 0)
    def _():
        s_ref[...] = jnp.zeros((8, 128), _f32)

    s_ref[...] += contrib


def _conv1_stats(A1, Bg):
    return pl.pallas_call(
        _stats_body,
        grid=(NB,),
        in_specs=[
            pl.BlockSpec((R, 128), lambda i: (i, 0)),
            pl.BlockSpec((K, R, 128), lambda i: (0, i, 0)),
        ],
        out_specs=pl.BlockSpec((8, 128), lambda i: (0, 0)),
        out_shape=jax.ShapeDtypeStruct((8, 128), _f32),
    )(A1, Bg)


def _c1fin_body(a_ref, bg_ref, s_ref, g_ref, be_ref, w_ref, b1b_ref, x1_ref):
    cnt = float(N * K)
    s = s_ref[0:1, :]
    ss = s_ref[1:2, :]
    mu = s / cnt
    var = ss / cnt - mu * mu
    alpha = g_ref[...] / jnp.sqrt(var + 1e-5)
    delta = be_ref[...] - mu * alpha
    w = w_ref[...]
    A = a_ref[...]
    m = None
    for k in range(K):
        h = jnp.maximum(alpha * (A + bg_ref[k]) + delta, 0.0)
        t = jnp.dot(h, w, preferred_element_type=_f32, precision=_PH)
        m = t if m is None else jnp.maximum(m, t)
    x1_ref[...] = m + b1b_ref[...]


def _conv1_fin(A1, Bg, sums, gamma, beta, W1b, b1b):
    return pl.pallas_call(
        _c1fin_body,
        grid=(NB,),
        in_specs=[
            pl.BlockSpec((R, 128), lambda i: (i, 0)),
            pl.BlockSpec((K, R, 128), lambda i: (0, i, 0)),
            pl.BlockSpec((8, 128), lambda i: (0, 0)),
            pl.BlockSpec((1, 128), lambda i: (0, 0)),
            pl.BlockSpec((1, 128), lambda i: (0, 0)),
            pl.BlockSpec((128, 64), lambda i: (0, 0)),
            pl.BlockSpec((1, 64), lambda i: (0, 0)),
        ],
        out_specs=pl.BlockSpec((R, 64), lambda i: (i, 0)),
        out_shape=jax.ShapeDtypeStruct((N, 64), _f32),
    )(A1, Bg, sums, gamma, beta, W1b, b1b)


def _bfin_body(q2_ref, rg_ref, x1_ref, wa_ref, wb_ref, bl_ref, bc_ref, h_ref):
    m = rg_ref[0]
    for k in range(1, K):
        m = jnp.maximum(m, rg_ref[k])
    x2 = q2_ref[...] + m
    hp = (jnp.dot(x1_ref[...], wa_ref[...], preferred_element_type=_f32, precision=_PH)
          + jnp.dot(x2, wb_ref[...], preferred_element_type=_f32, precision=_PH)
          + bl_ref[...])
    b = bc_ref[...]
    rows = []
    for g in range(G):
        vg = jnp.where(b == g, hp, -jnp.inf)
        rows.append(jnp.max(vg, axis=0, keepdims=True))
    contrib = jnp.concatenate(rows, axis=0)

    @pl.when(pl.program_id(0) == 0)
    def _():
        h_ref[...] = jnp.full((G, 128), -jnp.inf, _f32)

    h_ref[...] = jnp.maximum(h_ref[...], contrib)


def _branch_fin(q2, Rg, x1, Wlin_a, Wlin_b, blin, bcol):
    return pl.pallas_call(
        _bfin_body,
        grid=(NB,),
        in_specs=[
            pl.BlockSpec((R, 128), lambda i: (i, 0)),
            pl.BlockSpec((K, R, 128), lambda i: (0, i, 0)),
            pl.BlockSpec((R, 64), lambda i: (i, 0)),
            pl.BlockSpec((64, 128), lambda i: (0, 0)),
            pl.BlockSpec((128, 128), lambda i: (0, 0)),
            pl.BlockSpec((1, 128), lambda i: (0, 0)),
            pl.BlockSpec((R, 1), lambda i: (i, 0)),
        ],
        out_specs=pl.BlockSpec((G, 128), lambda i: (0, 0)),
        out_shape=jax.ShapeDtypeStruct((G, 128), _f32),
    )(q2, Rg, x1, Wlin_a, Wlin_b, blin, bcol)


def _mlp_body(h_ref, w1_ref, b1_ref, w2_ref, b2_ref, o_ref):
    t = jnp.maximum(jnp.dot(h_ref[...], w1_ref[...],
                            preferred_element_type=_f32, precision=_PH) + b1_ref[...], 0.0)
    o_ref[...] = jnp.dot(t, w2_ref[...],
                         preferred_element_type=_f32, precision=_PH) + b2_ref[...]


def _mlp(hh, Wm1, bm1, Wm2, bm2):
    return pl.pallas_call(
        _mlp_body,
        out_shape=jax.ShapeDtypeStruct((2 * G, 32), _f32),
    )(hh, Wm1, bm1, Wm2, bm2)


def kernel(pos_1, pos_2, batch, W1a, b1a, gamma1, beta1, W1b, b1b, W2, b2,
           Wlin, blin, Wm1, bm1, Wm2, bm2):
    batch = batch.astype(_i32)
    bcol = batch.reshape(N, 1)
    brow = batch.reshape(1, N)
    pad64 = lambda a: jnp.pad(a, ((0, 0), (0, 64)))
    W1ad = pad64(W1a[3:])
    W1aq = pad64(W1a[:3] - W1a[3:])
    b1a_p = jnp.pad(b1a, (0, 64)).reshape(1, -1)
    gamma_p = jnp.pad(gamma1, (0, 64)).reshape(1, -1)
    beta_p = jnp.pad(beta1, (0, 64)).reshape(1, -1)
    W1b_p = jnp.pad(W1b, ((0, 64), (0, 0)))
    W2d = W2[64:]
    W2q = W2[:64] - W2d
    Wlin_a = Wlin[:64]
    Wlin_b = Wlin[64:]

    def branch(p):
        idx1, A1, B1 = _knn_feats(p, bcol, brow, W1aq, b1a_p, W1ad)
        Bg = _sc_gather(B1, idx1.T.reshape(-1)).reshape(K, N, 128)
        sums = _conv1_stats(A1, Bg)
        x1 = _conv1_fin(A1, Bg, sums, gamma_p, beta_p, W1b_p,
                        b1b.reshape(1, -1))
        idx2, q2, R2 = _knn_feats(x1, bcol, brow, W2q, b2.reshape(1, -1), W2d)
        Rg = _sc_gather(R2, idx2.T.reshape(-1)).reshape(K, N, 128)
        return _branch_fin(q2, Rg, x1, Wlin_a, Wlin_b,
                           blin.reshape(1, -1), bcol)

    h1 = branch(pos_1)
    h2 = branch(pos_2)
    mm = _mlp(jnp.concatenate([h1, h2], axis=0), Wm1, bm1.reshape(1, -1),
              Wm2, bm2.reshape(1, -1))
    return h1, h2, mm[:G], mm[G:]
